# baseline (speedup 1.0000x reference)
import jax
import jax.numpy as jnp
import numpy as np
from jax import lax
from jax.experimental import pallas as pl
from jax.experimental.pallas import tpu as pltpu
from jax.experimental.pallas import tpu_sc as plsc

_BN_INV = float(1.0 / np.sqrt(1.0 + 1e-5))

N = 50000
D_IN = 39
EMB = 64
ROW = 128
E_LINK = 800000
E_INT = 400000
E_A2B = 400000
B_ALL = 2 * (E_LINK + E_INT + E_A2B)

NW = 32
CE = 200

BN = 2000
BE = 2000


_PCOL0, _SCOL, _VCOL0 = EMB, EMB + 3, EMB + 4


def _geo_selectors():
    col = lax.broadcasted_iota(jnp.int32, (ROW, 1), 0)
    sel_s = (col == _SCOL).astype(jnp.float32)
    sel_p = jnp.where((col >= _PCOL0) & (col < _PCOL0 + 3),
                      jnp.float32(-2.0), jnp.float32(0.0))
    sel_v = ((col >= _VCOL0) & (col < _VCOL0 + 3)).astype(jnp.float32)
    return sel_s, sel_p, sel_v


def _dot(a, b):
    return jnp.dot(a, b, preferred_element_type=jnp.float32,
                   precision=jax.lax.Precision.HIGHEST)


def _node_kernel(x_ref, xyz_ref, vec_ref,
                 w1_ref, b1_ref, g_ref, bt_ref, w2_ref, b2_ref,
                 nw1_ref, nb1_ref, ng_ref, nbt_ref, nw2_ref, nb2_ref,
                 tab_ref, node_ref):
    xb = x_ref[...]
    h = jnp.maximum(_dot(xb, w1_ref[...]) + b1_ref[...], 0.0)
    h = g_ref[...] * (h * _BN_INV) + bt_ref[...]
    e = jnp.maximum(_dot(h, w2_ref[...]) + b2_ref[...], 0.0)
    hn = jnp.maximum(_dot(e, nw1_ref[...]) + nb1_ref[...], 0.0)
    hn = ng_ref[...] * (hn * _BN_INV) + nbt_ref[...]
    node_ref[...] = _dot(hn, nw2_ref[...]) + nb2_ref[...]
    p = xyz_ref[...]
    v = vec_ref[...]
    s = jnp.sum(p * p, axis=1, keepdims=True)
    nrm = jnp.sqrt(jnp.sum(v * v, axis=1, keepdims=True))
    vh = v / jnp.maximum(nrm, 1e-8)
    tab_ref[...] = jnp.concatenate(
        [e, p, s, vh, jnp.zeros((p.shape[0], ROW - EMB - 7), jnp.float32)],
        axis=1)


def _full(shape):
    return pl.BlockSpec(shape, lambda i: tuple(0 for _ in shape))


def _node_stage(x, xyz, vec, fh_W1, fh_b1, fh_g, fh_bt, fh_W2, fh_b2,
                nd_W1, nd_b1, nd_g, nd_bt, nd_W2, nd_b2):
    grid = (N // BN,)
    return pl.pallas_call(
        _node_kernel,
        grid=grid,
        in_specs=[
            pl.BlockSpec((BN, D_IN), lambda i: (i, 0)),
            pl.BlockSpec((BN, 3), lambda i: (i, 0)),
            pl.BlockSpec((BN, 3), lambda i: (i, 0)),
            _full((D_IN, 256)), _full((256,)), _full((256,)), _full((256,)),
            _full((256, EMB)), _full((EMB,)),
            _full((EMB, 128)), _full((128,)), _full((128,)), _full((128,)),
            _full((128, 16)), _full((16,)),
        ],
        out_specs=[
            pl.BlockSpec((BN, ROW), lambda i: (i, 0)),
            pl.BlockSpec((BN, 16), lambda i: (i, 0)),
        ],
        out_shape=[
            jax.ShapeDtypeStruct((N, ROW), jnp.float32),
            jax.ShapeDtypeStruct((N, 16), jnp.float32),
        ],
    )(x, xyz, vec, fh_W1, fh_b1, fh_g, fh_bt, fh_W2, fh_b2,
      nd_W1, nd_b1, nd_g, nd_bt, nd_W2, nd_b2)


def _gather_call(tab, idx_all, row_off, n_rows):
    rpw = n_rows // NW
    npair = rpw // (2 * CE)
    tail = (rpw - npair * 2 * CE) // CE

    def body(tab_hbm, idx_hbm, out_hbm,
             idx0, idx1, rows0, rows1, sem0, sem1, ssem0, ssem1):
        wid = lax.axis_index("s") * 2 + lax.axis_index("c")
        base = wid * rpw

        def pair(i, carry):
            off0 = base + 2 * i * CE
            off1 = off0 + CE
            pltpu.sync_copy(idx_hbm.at[pl.ds(row_off + off0, CE)], idx0)
            g0 = pltpu.async_copy(tab_hbm.at[idx0], rows0, sem0)
            pltpu.sync_copy(idx_hbm.at[pl.ds(row_off + off1, CE)], idx1)
            g1 = pltpu.async_copy(tab_hbm.at[idx1], rows1, sem1)
            g0.wait()
            s0 = pltpu.async_copy(rows0, out_hbm.at[pl.ds(off0, CE)], ssem0)
            g1.wait()
            s1 = pltpu.async_copy(rows1, out_hbm.at[pl.ds(off1, CE)], ssem1)
            s0.wait()
            s1.wait()
            return carry

        lax.fori_loop(0, npair, pair, 0)
        if tail:
            off0 = base + npair * 2 * CE
            pltpu.sync_copy(idx_hbm.at[pl.ds(row_off + off0, CE)], idx0)
            pltpu.async_copy(tab_hbm.at[idx0], rows0, sem0).wait()
            pltpu.sync_copy(rows0, out_hbm.at[pl.ds(off0, CE)])

    mesh = plsc.VectorSubcoreMesh(core_axis_name="c", subcore_axis_name="s")
    k = pl.kernel(
        body,
        out_type=jax.ShapeDtypeStruct((n_rows, ROW), jnp.float32),
        mesh=mesh,
        scratch_types=[
            pltpu.VMEM((CE,), jnp.int32),
            pltpu.VMEM((CE,), jnp.int32),
            pltpu.VMEM((CE, ROW), jnp.float32),
            pltpu.VMEM((CE, ROW), jnp.float32),
            pltpu.SemaphoreType.DMA,
            pltpu.SemaphoreType.DMA,
            pltpu.SemaphoreType.DMA,
            pltpu.SemaphoreType.DMA,
        ],
    )
    return k(tab, idx_all)


def _pair_head_kernel(b0_ref, b1_ref, w1a_ref, w1b_ref, wd_ref, wa_ref,
                      bias1_ref, g_ref, bt_ref, w2_ref, b2_ref, out_ref):
    b0 = b0_ref[...]
    b1 = b1_ref[...]
    m = b0 * b1
    sel_s, sel_p, sel_v = _geo_selectors()
    d = _dot(b0 + b1, sel_s) + _dot(m, sel_p)
    a = _dot(m, sel_v)
    pre = (_dot(b0[:, :EMB], w1a_ref[...]) + _dot(b1[:, :EMB], w1b_ref[...])
           + d * wd_ref[...] + a * wa_ref[...] + bias1_ref[...])
    h = jnp.maximum(pre, 0.0)
    h = g_ref[...] * (h * _BN_INV) + bt_ref[...]
    out_ref[...] = _dot(h, w2_ref[...]) + b2_ref[...]


def _pair_head(T, n_edges, src_off, dst_off, W1, b1, g, bt, W2, b2,
               out_dim):
    w1a = W1[:EMB]
    w1b = W1[EMB:2 * EMB]
    wd = W1[2 * EMB:2 * EMB + 1]
    wa = W1[2 * EMB + 1:2 * EMB + 2]
    hid = W1.shape[1]
    grid = (n_edges // BE,)
    return pl.pallas_call(
        _pair_head_kernel,
        grid=grid,
        in_specs=[
            pl.BlockSpec((BE, ROW), lambda i: (i + src_off, 0)),
            pl.BlockSpec((BE, ROW), lambda i: (i + dst_off, 0)),
            _full((EMB, hid)), _full((EMB, hid)),
            _full((1, hid)), _full((1, hid)), _full((hid,)),
            _full((hid,)), _full((hid,)), _full((hid, out_dim)),
            _full((out_dim,)),
        ],
        out_specs=pl.BlockSpec((BE, out_dim), lambda i: (i, 0)),
        out_shape=jax.ShapeDtypeStruct((n_edges, out_dim), jnp.float32),
    )(T, T, w1a, w1b, wd, wa, b1, g, bt, W2, b2)


def _a2b_head_kernel(b0_ref, b1_ref, w1a_ref, w1b_ref, bias1_ref,
                     g_ref, bt_ref, w2_ref, b2_ref, out_ref):
    pre = (_dot(b0_ref[...][:, :EMB], w1a_ref[...])
           + _dot(b1_ref[...][:, :EMB], w1b_ref[...]) + bias1_ref[...])
    h = jnp.maximum(pre, 0.0)
    h = g_ref[...] * (h * _BN_INV) + bt_ref[...]
    out_ref[...] = _dot(h, w2_ref[...]) + b2_ref[...]


def _a2b_head(T, W1, b1, g, bt, W2, b2):
    w1a = W1[:EMB]
    w1b = W1[EMB:]
    hid = W1.shape[1]
    out_dim = W2.shape[1]
    dst_off = E_A2B // BE
    grid = (E_A2B // BE,)
    return pl.pallas_call(
        _a2b_head_kernel,
        grid=grid,
        in_specs=[
            pl.BlockSpec((BE, ROW), lambda i: (i, 0)),
            pl.BlockSpec((BE, ROW), lambda i: (i + dst_off, 0)),
            _full((EMB, hid)), _full((EMB, hid)), _full((hid,)),
            _full((hid,)), _full((hid,)), _full((hid, out_dim)),
            _full((out_dim,)),
        ],
        out_specs=pl.BlockSpec((BE, out_dim), lambda i: (i, 0)),
        out_shape=jax.ShapeDtypeStruct((E_A2B, out_dim), jnp.float32),
    )(T, T, w1a, w1b, b1, g, bt, W2, b2)


def kernel(x, edge_index, edge_attr, interaction_edge_index_pos,
           interaction_edge_index, xyz_data, vector_data, a2b_index, mask,
           fh_W1, fh_b1, fh_g, fh_bt, fh_W2, fh_b2,
           lc_W1, lc_b1, lc_g, lc_bt, lc_W2, lc_b2,
           ab_W1, ab_b1, ab_g, ab_bt, ab_W2, ab_b2,
           nd_W1, nd_b1, nd_g, nd_bt, nd_W2, nd_b2,
           it_W1, it_b1, it_g, it_bt, it_W2, it_b2):
    tab, node_preds = _node_stage(
        x, xyz_data, vector_data, fh_W1, fh_b1, fh_g, fh_bt, fh_W2, fh_b2,
        nd_W1, nd_b1, nd_g, nd_bt, nd_W2, nd_b2)

    idx_all = jnp.concatenate([
        interaction_edge_index[0], interaction_edge_index[1],
        interaction_edge_index_pos[0], interaction_edge_index_pos[1],
        a2b_index[0], a2b_index[1],
    ]).astype(jnp.int32)

    T_link = _gather_call(tab, idx_all, 0, 2 * E_LINK)
    T_int = _gather_call(tab, idx_all, 2 * E_LINK, 2 * E_INT)
    T_a2b = _gather_call(tab, idx_all, 2 * (E_LINK + E_INT), 2 * E_A2B)

    link_preds = _pair_head(T_link, E_LINK, 0, E_LINK // BE,
                            lc_W1, lc_b1, lc_g, lc_bt, lc_W2, lc_b2, 1)
    int_preds = _pair_head(T_int, E_INT, 0, E_INT // BE,
                           it_W1, it_b1, it_g, it_bt, it_W2, it_b2, 3)
    a2b_preds = _a2b_head(T_a2b, ab_W1, ab_b1, ab_g, ab_bt, ab_W2, ab_b2)

    return (link_preds, a2b_preds, node_preds, int_preds)


# speedup vs baseline: 1.4923x; 1.4923x over previous
import jax
import jax.numpy as jnp
import numpy as np
from jax import lax
from jax.experimental import pallas as pl
from jax.experimental.pallas import tpu as pltpu
from jax.experimental.pallas import tpu_sc as plsc

_BN_INV = float(1.0 / np.sqrt(1.0 + 1e-5))

N = 50000
D_IN = 39
EMB = 64
ROW = 128
E_LINK = 800000
E_INT = 400000
E_A2B = 400000
B_ALL = 2 * (E_LINK + E_INT + E_A2B)

NW = 32
CE = 200

BN = 2000
BE = 2000


_PCOL0, _SCOL, _VCOL0 = EMB, EMB + 3, EMB + 4


def _dot(a, b):
    return jnp.dot(a, b, preferred_element_type=jnp.float32,
                   precision=jax.lax.Precision.HIGHEST)


def _split(x):
    xh = x.astype(jnp.bfloat16)
    xl = (x - xh.astype(jnp.float32)).astype(jnp.bfloat16)
    return xh, xl


def _dot3(xh, xl, wh, wl):
    f = jnp.float32
    return (jnp.dot(xh, wh, preferred_element_type=f)
            + jnp.dot(xh, wl, preferred_element_type=f)
            + jnp.dot(xl, wh, preferred_element_type=f))


def _split_w(w):
    wh = w.astype(jnp.bfloat16)
    wl = (w - wh.astype(jnp.float32)).astype(jnp.bfloat16)
    return wh, wl


def _node_kernel(x_ref, xyz_ref, vec_ref,
                 w1_ref, b1_ref, g_ref, bt_ref, w2_ref, b2_ref,
                 nw1_ref, nb1_ref, ng_ref, nbt_ref, nw2_ref, nb2_ref,
                 tab_ref, node_ref):
    xb = x_ref[...]
    h = jnp.maximum(_dot(xb, w1_ref[...]) + b1_ref[...], 0.0)
    h = g_ref[...] * (h * _BN_INV) + bt_ref[...]
    e = jnp.maximum(_dot(h, w2_ref[...]) + b2_ref[...], 0.0)
    hn = jnp.maximum(_dot(e, nw1_ref[...]) + nb1_ref[...], 0.0)
    hn = ng_ref[...] * (hn * _BN_INV) + nbt_ref[...]
    node_ref[...] = _dot(hn, nw2_ref[...]) + nb2_ref[...]
    p = xyz_ref[...]
    v = vec_ref[...]
    s = jnp.sum(p * p, axis=1, keepdims=True)
    nrm = jnp.sqrt(jnp.sum(v * v, axis=1, keepdims=True))
    vh = v / jnp.maximum(nrm, 1e-8)
    tab_ref[...] = jnp.concatenate(
        [e, p, s, vh, jnp.zeros((p.shape[0], ROW - EMB - 7), jnp.float32)],
        axis=1)


def _full(shape):
    return pl.BlockSpec(shape, lambda i: tuple(0 for _ in shape))


def _node_stage(x, xyz, vec, fh_W1, fh_b1, fh_g, fh_bt, fh_W2, fh_b2,
                nd_W1, nd_b1, nd_g, nd_bt, nd_W2, nd_b2):
    grid = (N // BN,)
    return pl.pallas_call(
        _node_kernel,
        grid=grid,
        in_specs=[
            pl.BlockSpec((BN, D_IN), lambda i: (i, 0)),
            pl.BlockSpec((BN, 3), lambda i: (i, 0)),
            pl.BlockSpec((BN, 3), lambda i: (i, 0)),
            _full((D_IN, 256)), _full((256,)), _full((256,)), _full((256,)),
            _full((256, EMB)), _full((EMB,)),
            _full((EMB, 128)), _full((128,)), _full((128,)), _full((128,)),
            _full((128, 16)), _full((16,)),
        ],
        out_specs=[
            pl.BlockSpec((BN, ROW), lambda i: (i, 0)),
            pl.BlockSpec((BN, 16), lambda i: (i, 0)),
        ],
        out_shape=[
            jax.ShapeDtypeStruct((N, ROW), jnp.float32),
            jax.ShapeDtypeStruct((N, 16), jnp.float32),
        ],
    )(x, xyz, vec, fh_W1, fh_b1, fh_g, fh_bt, fh_W2, fh_b2,
      nd_W1, nd_b1, nd_g, nd_bt, nd_W2, nd_b2)


def _gather_call(tab, idx_all, row_off, n_rows):
    rpw = n_rows // NW
    npair = rpw // (2 * CE)
    tail = (rpw - npair * 2 * CE) // CE

    def body(tab_hbm, idx_hbm, out_hbm,
             idx0, idx1, rows0, rows1, sem0, sem1, ssem0, ssem1):
        wid = lax.axis_index("s") * 2 + lax.axis_index("c")
        base = wid * rpw

        def pair(i, carry):
            off0 = base + 2 * i * CE
            off1 = off0 + CE
            pltpu.sync_copy(idx_hbm.at[pl.ds(row_off + off0, CE)], idx0)
            g0 = pltpu.async_copy(tab_hbm.at[idx0], rows0, sem0)
            pltpu.sync_copy(idx_hbm.at[pl.ds(row_off + off1, CE)], idx1)
            g1 = pltpu.async_copy(tab_hbm.at[idx1], rows1, sem1)
            g0.wait()
            s0 = pltpu.async_copy(rows0, out_hbm.at[pl.ds(off0, CE)], ssem0)
            g1.wait()
            s1 = pltpu.async_copy(rows1, out_hbm.at[pl.ds(off1, CE)], ssem1)
            s0.wait()
            s1.wait()
            return carry

        lax.fori_loop(0, npair, pair, 0)
        if tail:
            off0 = base + npair * 2 * CE
            pltpu.sync_copy(idx_hbm.at[pl.ds(row_off + off0, CE)], idx0)
            pltpu.async_copy(tab_hbm.at[idx0], rows0, sem0).wait()
            pltpu.sync_copy(rows0, out_hbm.at[pl.ds(off0, CE)])

    mesh = plsc.VectorSubcoreMesh(core_axis_name="c", subcore_axis_name="s")
    k = pl.kernel(
        body,
        out_type=jax.ShapeDtypeStruct((n_rows, ROW), jnp.float32),
        mesh=mesh,
        scratch_types=[
            pltpu.VMEM((CE,), jnp.int32),
            pltpu.VMEM((CE,), jnp.int32),
            pltpu.VMEM((CE, ROW), jnp.float32),
            pltpu.VMEM((CE, ROW), jnp.float32),
            pltpu.SemaphoreType.DMA,
            pltpu.SemaphoreType.DMA,
            pltpu.SemaphoreType.DMA,
            pltpu.SemaphoreType.DMA,
        ],
    )
    return k(tab, idx_all)


def _pair_head_kernel(b0_ref, b1_ref, w1ah_ref, w1al_ref, w1bh_ref,
                      w1bl_ref, wgh_ref, wgl_ref, bias1_ref,
                      g_ref, bt_ref, w2h_ref, w2l_ref, b2_ref, out_ref):
    b0 = b0_ref[...]
    b1 = b1_ref[...]
    m = b0 * b1
    b0h, b0l = _split(b0)
    b1h, b1l = _split(b1)
    mh, ml = _split(m)
    pre = (_dot3(b0h, b0l, w1ah_ref[...], w1al_ref[...])
           + _dot3(b1h, b1l, w1bh_ref[...], w1bl_ref[...])
           + _dot3(mh, ml, wgh_ref[...], wgl_ref[...])
           + bias1_ref[...])
    h = jnp.maximum(pre, 0.0)
    h = g_ref[...] * (h * _BN_INV) + bt_ref[...]
    hh, hl = _split(h)
    out_ref[...] = _dot3(hh, hl, w2h_ref[...], w2l_ref[...]) + b2_ref[...]


def _pair_weights(W1):
    hid = W1.shape[1]
    wd = W1[2 * EMB]
    wa = W1[2 * EMB + 1]
    w1a = jnp.zeros((ROW, hid), jnp.float32).at[:EMB].set(W1[:EMB])
    w1a = w1a.at[_SCOL].set(wd)
    w1b = jnp.zeros((ROW, hid), jnp.float32).at[:EMB].set(W1[EMB:2 * EMB])
    w1b = w1b.at[_SCOL].set(wd)
    wgeo = jnp.zeros((ROW, hid), jnp.float32)
    wgeo = wgeo.at[_PCOL0:_PCOL0 + 3].set(jnp.tile(-2.0 * wd, (3, 1)))
    wgeo = wgeo.at[_VCOL0:_VCOL0 + 3].set(jnp.tile(wa, (3, 1)))
    return w1a, w1b, wgeo


def _pair_head(T, n_edges, src_off, dst_off, W1, b1, g, bt, W2, b2,
               out_dim):
    w1a, w1b, wgeo = _pair_weights(W1)
    w1ah, w1al = _split_w(w1a)
    w1bh, w1bl = _split_w(w1b)
    wgh, wgl = _split_w(wgeo)
    w2h, w2l = _split_w(W2)
    hid = W1.shape[1]
    grid = (n_edges // BE,)
    return pl.pallas_call(
        _pair_head_kernel,
        grid=grid,
        in_specs=[
            pl.BlockSpec((BE, ROW), lambda i: (i + src_off, 0)),
            pl.BlockSpec((BE, ROW), lambda i: (i + dst_off, 0)),
            _full((ROW, hid)), _full((ROW, hid)),
            _full((ROW, hid)), _full((ROW, hid)),
            _full((ROW, hid)), _full((ROW, hid)),
            _full((hid,)), _full((hid,)), _full((hid,)),
            _full((hid, out_dim)), _full((hid, out_dim)),
            _full((out_dim,)),
        ],
        out_specs=pl.BlockSpec((BE, out_dim), lambda i: (i, 0)),
        out_shape=jax.ShapeDtypeStruct((n_edges, out_dim), jnp.float32),
    )(T, T, w1ah, w1al, w1bh, w1bl, wgh, wgl, b1, g, bt, w2h, w2l, b2)


def _a2b_head_kernel(b0_ref, b1_ref, w1ah_ref, w1al_ref, w1bh_ref,
                     w1bl_ref, bias1_ref, g_ref, bt_ref, w2h_ref, w2l_ref,
                     b2_ref, out_ref):
    b0h, b0l = _split(b0_ref[...])
    b1h, b1l = _split(b1_ref[...])
    pre = (_dot3(b0h, b0l, w1ah_ref[...], w1al_ref[...])
           + _dot3(b1h, b1l, w1bh_ref[...], w1bl_ref[...])
           + bias1_ref[...])
    h = jnp.maximum(pre, 0.0)
    h = g_ref[...] * (h * _BN_INV) + bt_ref[...]
    hh, hl = _split(h)
    out_ref[...] = _dot3(hh, hl, w2h_ref[...], w2l_ref[...]) + b2_ref[...]


def _a2b_head(T, W1, b1, g, bt, W2, b2):
    hid = W1.shape[1]
    w1a = jnp.zeros((ROW, hid), jnp.float32).at[:EMB].set(W1[:EMB])
    w1b = jnp.zeros((ROW, hid), jnp.float32).at[:EMB].set(W1[EMB:])
    w1ah, w1al = _split_w(w1a)
    w1bh, w1bl = _split_w(w1b)
    w2h, w2l = _split_w(W2)
    out_dim = W2.shape[1]
    dst_off = E_A2B // BE
    grid = (E_A2B // BE,)
    return pl.pallas_call(
        _a2b_head_kernel,
        grid=grid,
        in_specs=[
            pl.BlockSpec((BE, ROW), lambda i: (i, 0)),
            pl.BlockSpec((BE, ROW), lambda i: (i + dst_off, 0)),
            _full((ROW, hid)), _full((ROW, hid)),
            _full((ROW, hid)), _full((ROW, hid)),
            _full((hid,)), _full((hid,)), _full((hid,)),
            _full((hid, out_dim)), _full((hid, out_dim)),
            _full((out_dim,)),
        ],
        out_specs=pl.BlockSpec((BE, out_dim), lambda i: (i, 0)),
        out_shape=jax.ShapeDtypeStruct((E_A2B, out_dim), jnp.float32),
    )(T, T, w1ah, w1al, w1bh, w1bl, b1, g, bt, w2h, w2l, b2)


def kernel(x, edge_index, edge_attr, interaction_edge_index_pos,
           interaction_edge_index, xyz_data, vector_data, a2b_index, mask,
           fh_W1, fh_b1, fh_g, fh_bt, fh_W2, fh_b2,
           lc_W1, lc_b1, lc_g, lc_bt, lc_W2, lc_b2,
           ab_W1, ab_b1, ab_g, ab_bt, ab_W2, ab_b2,
           nd_W1, nd_b1, nd_g, nd_bt, nd_W2, nd_b2,
           it_W1, it_b1, it_g, it_bt, it_W2, it_b2):
    tab, node_preds = _node_stage(
        x, xyz_data, vector_data, fh_W1, fh_b1, fh_g, fh_bt, fh_W2, fh_b2,
        nd_W1, nd_b1, nd_g, nd_bt, nd_W2, nd_b2)

    idx_all = jnp.concatenate([
        interaction_edge_index[0], interaction_edge_index[1],
        interaction_edge_index_pos[0], interaction_edge_index_pos[1],
        a2b_index[0], a2b_index[1],
    ]).astype(jnp.int32)

    T_link = _gather_call(tab, idx_all, 0, 2 * E_LINK)
    T_int = _gather_call(tab, idx_all, 2 * E_LINK, 2 * E_INT)
    T_a2b = _gather_call(tab, idx_all, 2 * (E_LINK + E_INT), 2 * E_A2B)

    link_preds = _pair_head(T_link, E_LINK, 0, E_LINK // BE,
                            lc_W1, lc_b1, lc_g, lc_bt, lc_W2, lc_b2, 1)
    int_preds = _pair_head(T_int, E_INT, 0, E_INT // BE,
                           it_W1, it_b1, it_g, it_bt, it_W2, it_b2, 3)
    a2b_preds = _a2b_head(T_a2b, ab_W1, ab_b1, ab_g, ab_bt, ab_W2, ab_b2)

    return (link_preds, a2b_preds, node_preds, int_preds)
